# baseline (speedup 1.0000x reference)
import functools

import jax
import jax.numpy as jnp
from jax import lax
from jax.experimental import pallas as pl
from jax.experimental.pallas import tpu as pltpu
from jax.experimental.pallas import tpu_sc as plsc

F = 64
F2 = 128
M = 16
NBR = 41
NCONV = 3
NCRYS = 512
EPS = 1e-5
NSPLIT = 25600


def _softplus(x):
    return jnp.maximum(x, 0.0) + jnp.log(1.0 + jnp.exp(-jnp.abs(x)))


def _sigmoid(x):
    return 0.5 + 0.5 * jnp.tanh(0.5 * x)


@functools.cache
def _make_gather(n_idx, feat):
    info = plsc.get_sparse_core_info()
    nc, ns = info.num_cores, info.num_subcores
    nw = nc * ns
    per_w = n_idx // nw
    assert per_w * nw == n_idx
    chb = 480
    n_full = per_w // chb
    tail = per_w - n_full * chb
    assert tail % 8 == 0 and n_full >= 2
    mesh = plsc.VectorSubcoreMesh(core_axis_name="c", subcore_axis_name="s")

    @functools.partial(
        pl.kernel,
        out_type=jax.ShapeDtypeStruct((n_idx, feat), jnp.float32),
        mesh=mesh,
        scratch_types=[
            pltpu.VMEM((chb,), jnp.int32),
            pltpu.VMEM((chb,), jnp.int32),
            pltpu.VMEM((chb, feat), jnp.float32),
            pltpu.VMEM((chb, feat), jnp.float32),
            pltpu.SemaphoreType.DMA,
        ],
    )
    def gather_k(y_hbm, idx_hbm, out_hbm, idx0, idx1, rows0, rows1, sem):
        wid = lax.axis_index("s") * nc + lax.axis_index("c")
        base = wid * per_w

        pltpu.sync_copy(idx_hbm.at[pl.ds(pl.multiple_of(base, 8), chb)], idx0)
        pltpu.async_copy(y_hbm.at[idx0], rows0, sem)

        def body(c, carry):
            b = lax.rem(c, 2)
            ofs = pl.multiple_of(base + c * chb, 8)
            pofs = pl.multiple_of(base + (c - 1) * chb, 8)

            @pl.when(b == 1)
            def _():
                pltpu.sync_copy(idx_hbm.at[pl.ds(ofs, chb)], idx1)
                pltpu.make_async_copy(y_hbm.at[idx0], rows0, sem).wait()
                pltpu.async_copy(y_hbm.at[idx1], rows1, sem)
                pltpu.sync_copy(rows0, out_hbm.at[pl.ds(pofs, chb)])

            @pl.when(b == 0)
            def _():
                pltpu.sync_copy(idx_hbm.at[pl.ds(ofs, chb)], idx0)
                pltpu.make_async_copy(y_hbm.at[idx1], rows1, sem).wait()
                pltpu.async_copy(y_hbm.at[idx0], rows0, sem)
                pltpu.sync_copy(rows1, out_hbm.at[pl.ds(pofs, chb)])

            return carry

        lax.fori_loop(1, n_full, body, 0)
        lofs = pl.multiple_of(base + (n_full - 1) * chb, 8)
        if (n_full - 1) % 2 == 0:
            l_idx, l_rows = idx0, rows0
            t_idx, t_rows = idx1, rows1
        else:
            l_idx, l_rows = idx1, rows1
            t_idx, t_rows = idx0, rows0
        pltpu.make_async_copy(y_hbm.at[l_idx], l_rows, sem).wait()
        pltpu.sync_copy(l_rows, out_hbm.at[pl.ds(lofs, chb)])
        if tail:
            tofs = pl.multiple_of(base + n_full * chb, 8)
            pltpu.sync_copy(idx_hbm.at[pl.ds(tofs, tail)],
                            t_idx.at[pl.ds(0, tail)])
            pltpu.async_copy(y_hbm.at[t_idx.at[pl.ds(0, tail)]],
                             t_rows.at[pl.ds(0, tail)], sem).wait()
            pltpu.sync_copy(t_rows.at[pl.ds(0, tail)],
                            out_hbm.at[pl.ds(tofs, tail)])

    return gather_k


def _emb(atom_fea, emb_W, emb_b, w_nbr, w_self, b):
    n, orig = atom_fea.shape
    nb = 2000

    def body(a_ref, w_ref, eb_ref, wn_ref, ws_ref, b_ref, x_ref, y_ref, z_ref):
        x = (jnp.dot(a_ref[...], w_ref[...], preferred_element_type=jnp.float32)
             + eb_ref[...])
        x_ref[...] = x
        y_ref[...] = jnp.dot(x, wn_ref[...], preferred_element_type=jnp.float32)
        z_ref[...] = (jnp.dot(x, ws_ref[...], preferred_element_type=jnp.float32)
                      + b_ref[...])

    return pl.pallas_call(
        body,
        grid=(n // nb,),
        in_specs=[
            pl.BlockSpec((nb, orig), lambda i: (i, 0)),
            pl.BlockSpec((orig, F), lambda i: (0, 0)),
            pl.BlockSpec((1, F), lambda i: (0, 0)),
            pl.BlockSpec((F, F2), lambda i: (0, 0)),
            pl.BlockSpec((F, F2), lambda i: (0, 0)),
            pl.BlockSpec((1, F2), lambda i: (0, 0)),
        ],
        out_specs=[
            pl.BlockSpec((nb, F), lambda i: (i, 0)),
            pl.BlockSpec((nb, F2), lambda i: (i, 0)),
            pl.BlockSpec((nb, F2), lambda i: (i, 0)),
        ],
        out_shape=[
            jax.ShapeDtypeStruct((n, F), jnp.float32),
            jax.ShapeDtypeStruct((n, F2), jnp.float32),
            jax.ShapeDtypeStruct((n, F2), jnp.float32),
        ],
    )(atom_fea, emb_W, emb_b.reshape(1, F), w_nbr, w_self, b)


def _pass_a(gath, z, nf3, w_edge, n, boff):
    nb = 400
    ne = nb * M

    def body(g_ref, z_ref, nf_ref, we_ref, gated_ref, st_ref):
        i = pl.program_id(0)
        e = jnp.dot(nf_ref[...].reshape(ne, NBR), we_ref[...],
                    preferred_element_type=jnp.float32)
        g2 = ((e + g_ref[...]).reshape(nb, M, F2)
              + z_ref[...][:, None, :]).reshape(ne, F2)
        gated_ref[...] = g2.astype(jnp.bfloat16)
        st = jnp.concatenate(
            [jnp.sum(g2, axis=0, keepdims=True),
             jnp.sum(g2 * g2, axis=0, keepdims=True)], axis=0)

        @pl.when(i == 0)
        def _():
            st_ref[...] = st

        @pl.when(i != 0)
        def _():
            st_ref[...] += st

    return pl.pallas_call(
        body,
        grid=(n // nb,),
        in_specs=[
            pl.BlockSpec((ne, F2), lambda i: (i, 0)),
            pl.BlockSpec((nb, F2), lambda i: (i + boff, 0)),
            pl.BlockSpec((nb, M, NBR), lambda i: (i + boff, 0, 0)),
            pl.BlockSpec((NBR, F2), lambda i: (0, 0)),
        ],
        out_specs=[
            pl.BlockSpec((ne, F2), lambda i: (i, 0)),
            pl.BlockSpec((2, F2), lambda i: (0, 0)),
        ],
        out_shape=[
            jax.ShapeDtypeStruct((n * M, F2), jnp.bfloat16),
            jax.ShapeDtypeStruct((2, F2), jnp.float32),
        ],
    )(gath, z, nf3, w_edge)


def _pass_b(gated, st, g1, b1, n):
    nb = 400
    ne = nb * M
    tot = float(50000 * M)

    def body(gd_ref, st_ref, g1_ref, b1_ref, ns_ref, st2_ref):
        i = pl.program_id(0)
        s = st_ref[...]
        mu = s[0:1, :] * (1.0 / tot)
        var = s[1:2, :] * (1.0 / tot) - mu * mu
        scale = g1_ref[...] * lax.rsqrt(var + EPS)
        shift = b1_ref[...] - mu * scale
        g2 = gd_ref[...].astype(jnp.float32) * scale + shift
        prod = _sigmoid(g2[:, :F]) * _softplus(g2[:, F:])
        ns = jnp.sum(prod.reshape(nb, M, F), axis=1)
        ns_ref[...] = ns
        st2 = jnp.concatenate(
            [jnp.sum(ns, axis=0, keepdims=True),
             jnp.sum(ns * ns, axis=0, keepdims=True)], axis=0)

        @pl.when(i == 0)
        def _():
            st2_ref[...] = st2

        @pl.when(i != 0)
        def _():
            st2_ref[...] += st2

    return pl.pallas_call(
        body,
        grid=(n // nb,),
        in_specs=[
            pl.BlockSpec((ne, F2), lambda i: (i, 0)),
            pl.BlockSpec((2, F2), lambda i: (0, 0)),
            pl.BlockSpec((1, F2), lambda i: (0, 0)),
            pl.BlockSpec((1, F2), lambda i: (0, 0)),
        ],
        out_specs=[
            pl.BlockSpec((nb, F), lambda i: (i, 0)),
            pl.BlockSpec((2, F), lambda i: (0, 0)),
        ],
        out_shape=[
            jax.ShapeDtypeStruct((n, F), jnp.float32),
            jax.ShapeDtypeStruct((2, F), jnp.float32),
        ],
    )(gated, st, g1, b1)


def _update(x, ns, st2, g2, b2, w_nbr, w_self, b):
    n = x.shape[0]
    nb = 2000

    def body(x_ref, ns_ref, st_ref, g_ref, bb_ref, wn_ref, ws_ref, b_ref,
             o_ref, y_ref, z_ref):
        s = st_ref[...]
        mu = s[0:1, :] * (1.0 / n)
        var = s[1:2, :] * (1.0 / n) - mu * mu
        scale = g_ref[...] * lax.rsqrt(var + EPS)
        shift = bb_ref[...] - mu * scale
        xn = _softplus(x_ref[...] + ns_ref[...] * scale + shift)
        o_ref[...] = xn
        y_ref[...] = jnp.dot(xn, wn_ref[...], preferred_element_type=jnp.float32)
        z_ref[...] = (jnp.dot(xn, ws_ref[...], preferred_element_type=jnp.float32)
                      + b_ref[...])

    return pl.pallas_call(
        body,
        grid=(n // nb,),
        in_specs=[
            pl.BlockSpec((nb, F), lambda i: (i, 0)),
            pl.BlockSpec((nb, F), lambda i: (i, 0)),
            pl.BlockSpec((2, F), lambda i: (0, 0)),
            pl.BlockSpec((1, F), lambda i: (0, 0)),
            pl.BlockSpec((1, F), lambda i: (0, 0)),
            pl.BlockSpec((F, F2), lambda i: (0, 0)),
            pl.BlockSpec((F, F2), lambda i: (0, 0)),
            pl.BlockSpec((1, F2), lambda i: (0, 0)),
        ],
        out_specs=[
            pl.BlockSpec((nb, F), lambda i: (i, 0)),
            pl.BlockSpec((nb, F2), lambda i: (i, 0)),
            pl.BlockSpec((nb, F2), lambda i: (i, 0)),
        ],
        out_shape=[
            jax.ShapeDtypeStruct((n, F), jnp.float32),
            jax.ShapeDtypeStruct((n, F2), jnp.float32),
            jax.ShapeDtypeStruct((n, F2), jnp.float32),
        ],
    )(x, ns, st2, g2, b2, w_nbr, w_self, b)


def _update_pool(x, ns, st2, g2, b2, seg):
    n = x.shape[0]
    nb = 2000

    def body(x_ref, ns_ref, st_ref, g_ref, b_ref, seg_ref, pool_ref, cnt_ref):
        i = pl.program_id(0)
        s = st_ref[...]
        mu = s[0:1, :] * (1.0 / n)
        var = s[1:2, :] * (1.0 / n) - mu * mu
        scale = g_ref[...] * lax.rsqrt(var + EPS)
        shift = b_ref[...] - mu * scale
        xn = _softplus(x_ref[...] + ns_ref[...] * scale + shift)
        oh = (seg_ref[...] == lax.broadcasted_iota(jnp.int32, (1, NCRYS), 1)
              ).astype(jnp.float32)
        pool = lax.dot_general(oh, xn, (((0,), (0,)), ((), ())),
                               preferred_element_type=jnp.float32)
        cnt = lax.dot_general(oh, jnp.ones((nb, 1), jnp.float32),
                              (((0,), (0,)), ((), ())),
                              preferred_element_type=jnp.float32)

        @pl.when(i == 0)
        def _():
            pool_ref[...] = pool
            cnt_ref[...] = cnt

        @pl.when(i != 0)
        def _():
            pool_ref[...] += pool
            cnt_ref[...] += cnt

    return pl.pallas_call(
        body,
        grid=(n // nb,),
        in_specs=[
            pl.BlockSpec((nb, F), lambda i: (i, 0)),
            pl.BlockSpec((nb, F), lambda i: (i, 0)),
            pl.BlockSpec((2, F), lambda i: (0, 0)),
            pl.BlockSpec((1, F), lambda i: (0, 0)),
            pl.BlockSpec((1, F), lambda i: (0, 0)),
            pl.BlockSpec((nb, 1), lambda i: (i, 0)),
        ],
        out_specs=[
            pl.BlockSpec((NCRYS, F), lambda i: (0, 0)),
            pl.BlockSpec((NCRYS, 1), lambda i: (0, 0)),
        ],
        out_shape=[
            jax.ShapeDtypeStruct((NCRYS, F), jnp.float32),
            jax.ShapeDtypeStruct((NCRYS, 1), jnp.float32),
        ],
    )(x, ns, st2, g2, b2, seg)


def _head(pool, cnt, w1, b1, w2, b2):
    def body(p_ref, c_ref, w1_ref, b1_ref, w2_ref, b2_ref, y_ref):
        crys = p_ref[...] / jnp.maximum(c_ref[...], 1.0)
        h = jnp.maximum(
            jnp.dot(crys, w1_ref[...], preferred_element_type=jnp.float32)
            + b1_ref[...], 0.0)
        y_ref[...] = (
            jnp.dot(h, w2_ref[...], preferred_element_type=jnp.float32)
            + b2_ref[...])

    return pl.pallas_call(
        body,
        out_shape=jax.ShapeDtypeStruct((NCRYS, F), jnp.float32),
    )(pool, cnt, w1, b1, w2, b2)


def kernel(atom_fea, nbr_fea, nbr_fea_idx, crystal_atom_idx,
           emb_W, emb_b, fc_W, fc_b, bn1_g, bn1_b, bn2_g, bn2_b,
           proj_W1, proj_b1, proj_W2, proj_b2):
    n, m = nbr_fea_idx.shape
    flat_idx = nbr_fea_idx.astype(jnp.int32).reshape(-1)
    nf_bf = nbr_fea.astype(jnp.bfloat16)
    seg = crystal_atom_idx.astype(jnp.int32).reshape(n, 1)

    x, y, z = _emb(atom_fea, emb_W, emb_b,
                   fc_W[0][F:2 * F], fc_W[0][:F], fc_b[0].reshape(1, F2))
    gather_fn = _make_gather(n * m, F2)

    idx_h1 = flat_idx[:NSPLIT * m]
    idx_h2 = flat_idx[NSPLIT * m:]
    n1, n2 = NSPLIT, n - NSPLIT
    gather1 = _make_gather(n1 * m, F2)
    gather2 = _make_gather(n2 * m, F2)

    pool = cnt = None
    for i in range(NCONV):
        w_edge = fc_W[i][2 * F:].astype(jnp.bfloat16)
        gath1 = gather1(y, idx_h1)
        gath2 = gather2(y, idx_h2)
        gated1, st_a = _pass_a(gath1, z, nf_bf, w_edge, n1, 0)
        gated2, st_b = _pass_a(gath2, z, nf_bf, w_edge, n2, NSPLIT // 400)
        st = st_a + st_b
        ns1, st2_a = _pass_b(gated1, st,
                             bn1_g[i].reshape(1, F2), bn1_b[i].reshape(1, F2),
                             n1)
        ns2, st2_b = _pass_b(gated2, st,
                             bn1_g[i].reshape(1, F2), bn1_b[i].reshape(1, F2),
                             n2)
        ns = jnp.concatenate([ns1, ns2], axis=0)
        st2 = st2_a + st2_b
        g2 = bn2_g[i].reshape(1, F)
        b2 = bn2_b[i].reshape(1, F)
        if i < NCONV - 1:
            x, y, z = _update(x, ns, st2, g2, b2,
                              fc_W[i + 1][F:2 * F], fc_W[i + 1][:F],
                              fc_b[i + 1].reshape(1, F2))
        else:
            pool, cnt = _update_pool(x, ns, st2, g2, b2, seg)

    return _head(pool, cnt, proj_W1, proj_b1.reshape(1, F),
                 proj_W2, proj_b2.reshape(1, F))


# speedup vs baseline: 1.0385x; 1.0385x over previous
import functools

import jax
import jax.numpy as jnp
from jax import lax
from jax.experimental import pallas as pl
from jax.experimental.pallas import tpu as pltpu
from jax.experimental.pallas import tpu_sc as plsc

F = 64
F2 = 128
M = 16
NBR = 41
NCONV = 3
NCRYS = 512
EPS = 1e-5


def _softplus(x):
    return jnp.maximum(x, 0.0) + jnp.log(1.0 + jnp.exp(-jnp.abs(x)))


def _sigmoid(x):
    return 0.5 + 0.5 * jnp.tanh(0.5 * x)


@functools.cache
def _make_gather(n_idx, feat):
    info = plsc.get_sparse_core_info()
    nc, ns = info.num_cores, info.num_subcores
    nw = nc * ns
    per_w = n_idx // nw
    assert per_w * nw == n_idx
    chb = 480
    n_full = per_w // chb
    tail = per_w - n_full * chb
    assert tail % 8 == 0 and n_full >= 2
    mesh = plsc.VectorSubcoreMesh(core_axis_name="c", subcore_axis_name="s")

    @functools.partial(
        pl.kernel,
        out_type=jax.ShapeDtypeStruct((n_idx, feat), jnp.float32),
        mesh=mesh,
        scratch_types=[
            pltpu.VMEM((chb,), jnp.int32),
            pltpu.VMEM((chb,), jnp.int32),
            pltpu.VMEM((chb, feat), jnp.float32),
            pltpu.VMEM((chb, feat), jnp.float32),
            pltpu.SemaphoreType.DMA,
        ],
    )
    def gather_k(y_hbm, idx_hbm, out_hbm, idx0, idx1, rows0, rows1, sem):
        wid = lax.axis_index("s") * nc + lax.axis_index("c")
        base = wid * per_w

        pltpu.sync_copy(idx_hbm.at[pl.ds(pl.multiple_of(base, 8), chb)], idx0)
        pltpu.async_copy(y_hbm.at[idx0], rows0, sem)

        def body(c, carry):
            b = lax.rem(c, 2)
            ofs = pl.multiple_of(base + c * chb, 8)
            pofs = pl.multiple_of(base + (c - 1) * chb, 8)

            @pl.when(b == 1)
            def _():
                pltpu.sync_copy(idx_hbm.at[pl.ds(ofs, chb)], idx1)
                pltpu.make_async_copy(y_hbm.at[idx0], rows0, sem).wait()
                pltpu.async_copy(y_hbm.at[idx1], rows1, sem)
                pltpu.sync_copy(rows0, out_hbm.at[pl.ds(pofs, chb)])

            @pl.when(b == 0)
            def _():
                pltpu.sync_copy(idx_hbm.at[pl.ds(ofs, chb)], idx0)
                pltpu.make_async_copy(y_hbm.at[idx1], rows1, sem).wait()
                pltpu.async_copy(y_hbm.at[idx0], rows0, sem)
                pltpu.sync_copy(rows1, out_hbm.at[pl.ds(pofs, chb)])

            return carry

        lax.fori_loop(1, n_full, body, 0)
        lofs = pl.multiple_of(base + (n_full - 1) * chb, 8)
        if (n_full - 1) % 2 == 0:
            l_idx, l_rows = idx0, rows0
            t_idx, t_rows = idx1, rows1
        else:
            l_idx, l_rows = idx1, rows1
            t_idx, t_rows = idx0, rows0
        pltpu.make_async_copy(y_hbm.at[l_idx], l_rows, sem).wait()
        pltpu.sync_copy(l_rows, out_hbm.at[pl.ds(lofs, chb)])
        if tail:
            tofs = pl.multiple_of(base + n_full * chb, 8)
            pltpu.sync_copy(idx_hbm.at[pl.ds(tofs, tail)],
                            t_idx.at[pl.ds(0, tail)])
            pltpu.async_copy(y_hbm.at[t_idx.at[pl.ds(0, tail)]],
                             t_rows.at[pl.ds(0, tail)], sem).wait()
            pltpu.sync_copy(t_rows.at[pl.ds(0, tail)],
                            out_hbm.at[pl.ds(tofs, tail)])

    return gather_k


def _emb(atom_fea, emb_W, emb_b, w_nbr, w_self, b):
    n, orig = atom_fea.shape
    nb = 2000

    def body(a_ref, w_ref, eb_ref, wn_ref, ws_ref, b_ref, x_ref, y_ref, z_ref):
        x = (jnp.dot(a_ref[...], w_ref[...], preferred_element_type=jnp.float32)
             + eb_ref[...])
        x_ref[...] = x
        y_ref[...] = jnp.dot(x, wn_ref[...], preferred_element_type=jnp.float32)
        z_ref[...] = (jnp.dot(x, ws_ref[...], preferred_element_type=jnp.float32)
                      + b_ref[...])

    return pl.pallas_call(
        body,
        grid=(n // nb,),
        in_specs=[
            pl.BlockSpec((nb, orig), lambda i: (i, 0)),
            pl.BlockSpec((orig, F), lambda i: (0, 0)),
            pl.BlockSpec((1, F), lambda i: (0, 0)),
            pl.BlockSpec((F, F2), lambda i: (0, 0)),
            pl.BlockSpec((F, F2), lambda i: (0, 0)),
            pl.BlockSpec((1, F2), lambda i: (0, 0)),
        ],
        out_specs=[
            pl.BlockSpec((nb, F), lambda i: (i, 0)),
            pl.BlockSpec((nb, F2), lambda i: (i, 0)),
            pl.BlockSpec((nb, F2), lambda i: (i, 0)),
        ],
        out_shape=[
            jax.ShapeDtypeStruct((n, F), jnp.float32),
            jax.ShapeDtypeStruct((n, F2), jnp.float32),
            jax.ShapeDtypeStruct((n, F2), jnp.float32),
        ],
    )(atom_fea, emb_W, emb_b.reshape(1, F), w_nbr, w_self, b)


def _pass_a(gath, z, nf3, w_edge, n):
    nb = 400
    ne = nb * M

    def body(g_ref, z_ref, nf_ref, we_ref, gated_ref, st_ref):
        i = pl.program_id(0)
        e = jnp.dot(nf_ref[...].reshape(ne, NBR), we_ref[...],
                    preferred_element_type=jnp.float32)
        g2 = ((e + g_ref[...]).reshape(nb, M, F2)
              + z_ref[...][:, None, :]).reshape(ne, F2)
        gated_ref[...] = g2.astype(jnp.bfloat16)
        st = jnp.concatenate(
            [jnp.sum(g2, axis=0, keepdims=True),
             jnp.sum(g2 * g2, axis=0, keepdims=True)], axis=0)

        @pl.when(i == 0)
        def _():
            st_ref[...] = st

        @pl.when(i != 0)
        def _():
            st_ref[...] += st

    return pl.pallas_call(
        body,
        grid=(n // nb,),
        in_specs=[
            pl.BlockSpec((ne, F2), lambda i: (i, 0)),
            pl.BlockSpec((nb, F2), lambda i: (i, 0)),
            pl.BlockSpec((nb, M, NBR), lambda i: (i, 0, 0)),
            pl.BlockSpec((NBR, F2), lambda i: (0, 0)),
        ],
        out_specs=[
            pl.BlockSpec((ne, F2), lambda i: (i, 0)),
            pl.BlockSpec((2, F2), lambda i: (0, 0)),
        ],
        out_shape=[
            jax.ShapeDtypeStruct((n * M, F2), jnp.bfloat16),
            jax.ShapeDtypeStruct((2, F2), jnp.float32),
        ],
    )(gath, z, nf3, w_edge)


def _pass_b(gated, st, g1, b1, n):
    nb = 1000
    ne = nb * M
    tot = float(n * M)

    def body(gd_ref, st_ref, g1_ref, b1_ref, ns_ref, st2_ref):
        i = pl.program_id(0)
        s = st_ref[...]
        mu = s[0:1, :] * (1.0 / tot)
        var = s[1:2, :] * (1.0 / tot) - mu * mu
        scale = g1_ref[...] * lax.rsqrt(var + EPS)
        shift = b1_ref[...] - mu * scale
        g2 = gd_ref[...].astype(jnp.float32) * scale + shift
        prod = _sigmoid(g2[:, :F]) * _softplus(g2[:, F:])
        ns = jnp.sum(prod.reshape(nb, M, F), axis=1)
        ns_ref[...] = ns
        st2 = jnp.concatenate(
            [jnp.sum(ns, axis=0, keepdims=True),
             jnp.sum(ns * ns, axis=0, keepdims=True)], axis=0)

        @pl.when(i == 0)
        def _():
            st2_ref[...] = st2

        @pl.when(i != 0)
        def _():
            st2_ref[...] += st2

    return pl.pallas_call(
        body,
        grid=(n // nb,),
        in_specs=[
            pl.BlockSpec((ne, F2), lambda i: (i, 0)),
            pl.BlockSpec((2, F2), lambda i: (0, 0)),
            pl.BlockSpec((1, F2), lambda i: (0, 0)),
            pl.BlockSpec((1, F2), lambda i: (0, 0)),
        ],
        out_specs=[
            pl.BlockSpec((nb, F), lambda i: (i, 0)),
            pl.BlockSpec((2, F), lambda i: (0, 0)),
        ],
        out_shape=[
            jax.ShapeDtypeStruct((n, F), jnp.float32),
            jax.ShapeDtypeStruct((2, F), jnp.float32),
        ],
    )(gated, st, g1, b1)


def _update(x, ns, st2, g2, b2, w_nbr, w_self, b):
    n = x.shape[0]
    nb = 2000

    def body(x_ref, ns_ref, st_ref, g_ref, bb_ref, wn_ref, ws_ref, b_ref,
             o_ref, y_ref, z_ref):
        s = st_ref[...]
        mu = s[0:1, :] * (1.0 / n)
        var = s[1:2, :] * (1.0 / n) - mu * mu
        scale = g_ref[...] * lax.rsqrt(var + EPS)
        shift = bb_ref[...] - mu * scale
        xn = _softplus(x_ref[...] + ns_ref[...] * scale + shift)
        o_ref[...] = xn
        y_ref[...] = jnp.dot(xn, wn_ref[...], preferred_element_type=jnp.float32)
        z_ref[...] = (jnp.dot(xn, ws_ref[...], preferred_element_type=jnp.float32)
                      + b_ref[...])

    return pl.pallas_call(
        body,
        grid=(n // nb,),
        in_specs=[
            pl.BlockSpec((nb, F), lambda i: (i, 0)),
            pl.BlockSpec((nb, F), lambda i: (i, 0)),
            pl.BlockSpec((2, F), lambda i: (0, 0)),
            pl.BlockSpec((1, F), lambda i: (0, 0)),
            pl.BlockSpec((1, F), lambda i: (0, 0)),
            pl.BlockSpec((F, F2), lambda i: (0, 0)),
            pl.BlockSpec((F, F2), lambda i: (0, 0)),
            pl.BlockSpec((1, F2), lambda i: (0, 0)),
        ],
        out_specs=[
            pl.BlockSpec((nb, F), lambda i: (i, 0)),
            pl.BlockSpec((nb, F2), lambda i: (i, 0)),
            pl.BlockSpec((nb, F2), lambda i: (i, 0)),
        ],
        out_shape=[
            jax.ShapeDtypeStruct((n, F), jnp.float32),
            jax.ShapeDtypeStruct((n, F2), jnp.float32),
            jax.ShapeDtypeStruct((n, F2), jnp.float32),
        ],
    )(x, ns, st2, g2, b2, w_nbr, w_self, b)


def _update_pool(x, ns, st2, g2, b2, seg):
    n = x.shape[0]
    nb = 2000

    def body(x_ref, ns_ref, st_ref, g_ref, b_ref, seg_ref, pool_ref, cnt_ref):
        i = pl.program_id(0)
        s = st_ref[...]
        mu = s[0:1, :] * (1.0 / n)
        var = s[1:2, :] * (1.0 / n) - mu * mu
        scale = g_ref[...] * lax.rsqrt(var + EPS)
        shift = b_ref[...] - mu * scale
        xn = _softplus(x_ref[...] + ns_ref[...] * scale + shift)
        oh = (seg_ref[...] == lax.broadcasted_iota(jnp.int32, (1, NCRYS), 1)
              ).astype(jnp.float32)
        pool = lax.dot_general(oh, xn, (((0,), (0,)), ((), ())),
                               preferred_element_type=jnp.float32)
        cnt = lax.dot_general(oh, jnp.ones((nb, 1), jnp.float32),
                              (((0,), (0,)), ((), ())),
                              preferred_element_type=jnp.float32)

        @pl.when(i == 0)
        def _():
            pool_ref[...] = pool
            cnt_ref[...] = cnt

        @pl.when(i != 0)
        def _():
            pool_ref[...] += pool
            cnt_ref[...] += cnt

    return pl.pallas_call(
        body,
        grid=(n // nb,),
        in_specs=[
            pl.BlockSpec((nb, F), lambda i: (i, 0)),
            pl.BlockSpec((nb, F), lambda i: (i, 0)),
            pl.BlockSpec((2, F), lambda i: (0, 0)),
            pl.BlockSpec((1, F), lambda i: (0, 0)),
            pl.BlockSpec((1, F), lambda i: (0, 0)),
            pl.BlockSpec((nb, 1), lambda i: (i, 0)),
        ],
        out_specs=[
            pl.BlockSpec((NCRYS, F), lambda i: (0, 0)),
            pl.BlockSpec((NCRYS, 1), lambda i: (0, 0)),
        ],
        out_shape=[
            jax.ShapeDtypeStruct((NCRYS, F), jnp.float32),
            jax.ShapeDtypeStruct((NCRYS, 1), jnp.float32),
        ],
    )(x, ns, st2, g2, b2, seg)


def _head(pool, cnt, w1, b1, w2, b2):
    def body(p_ref, c_ref, w1_ref, b1_ref, w2_ref, b2_ref, y_ref):
        crys = p_ref[...] / jnp.maximum(c_ref[...], 1.0)
        h = jnp.maximum(
            jnp.dot(crys, w1_ref[...], preferred_element_type=jnp.float32)
            + b1_ref[...], 0.0)
        y_ref[...] = (
            jnp.dot(h, w2_ref[...], preferred_element_type=jnp.float32)
            + b2_ref[...])

    return pl.pallas_call(
        body,
        out_shape=jax.ShapeDtypeStruct((NCRYS, F), jnp.float32),
    )(pool, cnt, w1, b1, w2, b2)


def kernel(atom_fea, nbr_fea, nbr_fea_idx, crystal_atom_idx,
           emb_W, emb_b, fc_W, fc_b, bn1_g, bn1_b, bn2_g, bn2_b,
           proj_W1, proj_b1, proj_W2, proj_b2):
    n, m = nbr_fea_idx.shape
    flat_idx = nbr_fea_idx.astype(jnp.int32).reshape(-1)
    nf_bf = nbr_fea.astype(jnp.bfloat16)
    seg = crystal_atom_idx.astype(jnp.int32).reshape(n, 1)

    x, y, z = _emb(atom_fea, emb_W, emb_b,
                   fc_W[0][F:2 * F], fc_W[0][:F], fc_b[0].reshape(1, F2))
    gather_fn = _make_gather(n * m, F2)

    pool = cnt = None
    for i in range(NCONV):
        w_edge = fc_W[i][2 * F:].astype(jnp.bfloat16)
        gath = gather_fn(y, flat_idx)
        gated, st = _pass_a(gath, z, nf_bf, w_edge, n)
        ns, st2 = _pass_b(gated, st,
                          bn1_g[i].reshape(1, F2), bn1_b[i].reshape(1, F2), n)
        g2 = bn2_g[i].reshape(1, F)
        b2 = bn2_b[i].reshape(1, F)
        if i < NCONV - 1:
            x, y, z = _update(x, ns, st2, g2, b2,
                              fc_W[i + 1][F:2 * F], fc_W[i + 1][:F],
                              fc_b[i + 1].reshape(1, F2))
        else:
            pool, cnt = _update_pool(x, ns, st2, g2, b2, seg)

    return _head(pool, cnt, proj_W1, proj_b1.reshape(1, F),
                 proj_W2, proj_b2.reshape(1, F))
